# baseline (speedup 1.0000x reference)
import functools

import jax
import jax.numpy as jnp
from jax import lax
from jax.experimental import pallas as pl
from jax.experimental.pallas import tpu as pltpu
from jax.experimental.pallas import tpu_sc as plsc

VOCAB = 1_000_000
H = 64
B = 16384
L = 20

NC = 2
NS = 16
NW = NC * NS
BPW = B // NW
CB = 32
NCH = BPW // CB
RPC = CB * L
NG = RPC // 128
HW = H // 2


def _sc_scores_body(center_hbm, ctx_hbm, u_hbm, v_hbm, out_hbm,
                    cidx, vidx, urows, vrows, sbuf, gsem):
    wid = lax.axis_index("s") * NC + lax.axis_index("c")

    def fire(g, slot):
        base = pl.multiple_of(wid * BPW + g * CB, CB)
        pltpu.sync_copy(center_hbm.at[pl.ds(base, CB)], cidx.at[slot])
        off = pl.multiple_of((wid * BPW + g * CB) * L, RPC)
        pltpu.sync_copy(ctx_hbm.at[pl.ds(off, RPC)], vidx.at[slot])
        pltpu.async_copy(u_hbm.at[cidx.at[slot]], urows.at[slot],
                         gsem.at[slot])
        for j in range(NG):
            pltpu.async_copy(v_hbm.at[vidx.at[slot, pl.ds(j * 128, 128)]],
                             vrows.at[slot, pl.ds(j * 128, 128)],
                             gsem.at[slot])

    def wait_gathers(slot):
        pltpu.make_async_copy(u_hbm.at[cidx.at[slot]], urows.at[slot],
                              gsem.at[slot]).wait()
        for j in range(NG):
            pltpu.make_async_copy(v_hbm.at[vidx.at[slot, pl.ds(j * 128, 128)]],
                                  vrows.at[slot, pl.ds(j * 128, 128)],
                                  gsem.at[slot]).wait()

    def compute(g, slot):
        iota16 = lax.iota(jnp.int32, 16)

        def bbody(b, carry):
            r_a = b * L + iota16
            r_b = r_a + 16
            acc_a = jnp.zeros((16,), jnp.float32)
            acc_b = jnp.zeros((16,), jnp.float32)
            uw0 = urows[slot, b, pl.ds(0, 16)]
            uw1 = urows[slot, b, pl.ds(16, 16)]
            ue0, uo0 = plsc.unpack(plsc.bitcast(uw0, jnp.bfloat16),
                                   format=plsc.PackFormat.INTERLEAVED)
            ue1, uo1 = plsc.unpack(plsc.bitcast(uw1, jnp.bfloat16),
                                   format=plsc.PackFormat.INTERLEAVED)
            for w in range(HW):
                col = jnp.full((16,), w, jnp.int32)
                g_a = plsc.load_gather(vrows.at[slot], [r_a, col])
                g_b = plsc.load_gather(vrows.at[slot], [r_b, col])
                va_e, va_o = plsc.unpack(plsc.bitcast(g_a, jnp.bfloat16),
                                         format=plsc.PackFormat.INTERLEAVED)
                vb_e, vb_o = plsc.unpack(plsc.bitcast(g_b, jnp.bfloat16),
                                         format=plsc.PackFormat.INTERLEAVED)
                ue = ue0 if w < 16 else ue1
                uo = uo0 if w < 16 else uo1
                u_even = ue[w % 16]
                u_odd = uo[w % 16]
                acc_a = acc_a + u_even * va_e + u_odd * va_o
                acc_b = acc_b + u_even * vb_e + u_odd * vb_o
            sbuf[slot, pl.ds(b * L, 16)] = acc_a
            sbuf[slot, pl.ds(b * L + 16, 16)] = acc_b
            return carry

        lax.fori_loop(0, CB, bbody, 0)
        base = pl.multiple_of((wid * BPW + g * CB) * L, RPC)
        pltpu.sync_copy(sbuf.at[slot, pl.ds(0, RPC)],
                        out_hbm.at[pl.ds(base, RPC)])

    fire(0, 0)

    def pair(i, carry):
        for s in (0, 1):
            g = i * 2 + s

            @pl.when(g + 1 < NCH)
            def _():
                fire(g + 1, (s + 1) % 2)

            wait_gathers(s)
            compute(g, s)
        return carry

    lax.fori_loop(0, NCH // 2, pair, 0)


def _sc_scores(center_flat, ctx_flat, u, v):
    mesh = plsc.VectorSubcoreMesh(core_axis_name="c", subcore_axis_name="s",
                                  num_cores=NC, num_subcores=NS)
    return pl.kernel(
        _sc_scores_body,
        out_type=jax.ShapeDtypeStruct((B * L,), jnp.float32),
        mesh=mesh,
        scratch_types=[
            pltpu.VMEM((2, CB), jnp.int32),
            pltpu.VMEM((2, RPC), jnp.int32),
            pltpu.VMEM((2, CB, HW), jnp.int32),
            pltpu.VMEM((2, RPC + 16, HW), jnp.int32),
            pltpu.VMEM((2, RPC + 32), jnp.float32),
            pltpu.SemaphoreType.DMA((2,)),
        ],
        compiler_params=pltpu.CompilerParams(needs_layout_passes=False,
                                             use_tc_tiling_on_sc=False),
    )(center_flat, ctx_flat, u, v)


def _loss_body(s_ref, lab_ref, m_ref, out_ref):
    s = s_ref[...]
    lab = lab_ref[...]
    m = m_ref[...]
    per = jnp.maximum(s, 0.0) - s * lab + jnp.log1p(jnp.exp(-jnp.abs(s)))
    num = jnp.sum(per * m)
    den = jnp.maximum(jnp.sum(m), 1.0)
    out_ref[0, 0] = num / den


def _tc_loss(scores2d, label2d, mask2d):
    return pl.pallas_call(
        _loss_body,
        out_shape=jax.ShapeDtypeStruct((1, 1), jnp.float32),
        out_specs=pl.BlockSpec(memory_space=pltpu.SMEM),
    )(scores2d, label2d, mask2d)


def _pack_bf16(table):
    t16 = table.astype(jnp.bfloat16).reshape(VOCAB, HW, 2)
    return jax.lax.bitcast_convert_type(t16, jnp.int32)


def kernel(center, context_neg, label, mask, U, V):
    center_flat = center.reshape(B)
    ctx_flat = context_neg.reshape(B * L)
    scores = _sc_scores(center_flat, ctx_flat, _pack_bf16(U), _pack_bf16(V))
    scores2d = scores.reshape(B * L // 128, 128)
    label2d = label.reshape(B * L // 128, 128)
    mask2d = mask.reshape(B * L // 128, 128)
    return _tc_loss(scores2d, label2d, mask2d).reshape(())


# speedup vs baseline: 2.2101x; 2.2101x over previous
import functools

import jax
import jax.numpy as jnp
from jax import lax
from jax.experimental import pallas as pl
from jax.experimental.pallas import tpu as pltpu
from jax.experimental.pallas import tpu_sc as plsc

VOCAB = 1_000_000
H = 64
B = 16384
L = 20

NC = 2
NS = 16
NW = NC * NS
BPW = B // NW
CB = 32
NCH = BPW // CB
RPC = CB * L
NG = RPC // 128


def _sc_scores_body(center_hbm, ctx_hbm, u_hbm, v_hbm, out_hbm,
                    cidx, vidx, urows, vrows, tbuf, sbuf, gsem):
    wid = lax.axis_index("s") * NC + lax.axis_index("c")
    iota16 = lax.iota(jnp.int32, 16)

    def fire(g, slot):
        base = pl.multiple_of(wid * BPW + g * CB, CB)
        pltpu.sync_copy(center_hbm.at[pl.ds(base, CB)], cidx.at[slot])
        off = pl.multiple_of((wid * BPW + g * CB) * L, RPC)
        pltpu.sync_copy(ctx_hbm.at[pl.ds(off, RPC)], vidx.at[slot])
        pltpu.async_copy(u_hbm.at[cidx.at[slot]], urows.at[slot],
                         gsem.at[slot])
        for j in range(NG):
            pltpu.async_copy(v_hbm.at[vidx.at[slot, pl.ds(j * 128, 128)]],
                             vrows.at[slot, pl.ds(j * 128, 128)],
                             gsem.at[slot])

    def wait_gathers(slot):
        pltpu.make_async_copy(u_hbm.at[cidx.at[slot]], urows.at[slot],
                              gsem.at[slot]).wait()
        for j in range(NG):
            pltpu.make_async_copy(v_hbm.at[vidx.at[slot, pl.ds(j * 128, 128)]],
                                  vrows.at[slot, pl.ds(j * 128, 128)],
                                  gsem.at[slot]).wait()

    def compute(g, slot):
        def hsum16(nrows):
            tot = jnp.zeros((16,), jnp.float32)
            for k in range(16):
                col = jnp.full((16,), k, jnp.int32)
                tot = tot + plsc.load_gather(tbuf, [iota16, col])
            return tot

        def bbody(b, carry):
            u0 = urows[slot, b, pl.ds(0, 32)]
            u1 = urows[slot, b, pl.ds(32, 32)]
            for l in range(16):
                r = b * L + l
                p = (vrows[slot, r, pl.ds(0, 32)] * u0
                     + vrows[slot, r, pl.ds(32, 32)] * u1)
                pe, po = plsc.unpack(p, format=plsc.PackFormat.INTERLEAVED)
                tbuf[l] = pe + po
            sbuf[slot, pl.ds(b * L, 16)] = hsum16(16)
            for l in range(16, L):
                r = b * L + l
                p = (vrows[slot, r, pl.ds(0, 32)] * u0
                     + vrows[slot, r, pl.ds(32, 32)] * u1)
                pe, po = plsc.unpack(p, format=plsc.PackFormat.INTERLEAVED)
                tbuf[l - 16] = pe + po
            sbuf[slot, pl.ds(b * L + 16, 16)] = hsum16(4)
            return carry

        lax.fori_loop(0, CB, bbody, 0)
        base = pl.multiple_of((wid * BPW + g * CB) * L, RPC)
        pltpu.sync_copy(sbuf.at[slot, pl.ds(0, RPC)],
                        out_hbm.at[pl.ds(base, RPC)])

    fire(0, 0)

    def pair(i, carry):
        for s in (0, 1):
            g = i * 2 + s

            @pl.when(g + 1 < NCH)
            def _():
                fire(g + 1, (s + 1) % 2)

            wait_gathers(s)
            compute(g, s)
        return carry

    lax.fori_loop(0, NCH // 2, pair, 0)


def _sc_scores(center_flat, ctx_flat, u16, v16):
    mesh = plsc.VectorSubcoreMesh(core_axis_name="c", subcore_axis_name="s",
                                  num_cores=NC, num_subcores=NS)
    return pl.kernel(
        _sc_scores_body,
        out_type=jax.ShapeDtypeStruct((B * L,), jnp.float32),
        mesh=mesh,
        scratch_types=[
            pltpu.VMEM((2, CB), jnp.int32),
            pltpu.VMEM((2, RPC), jnp.int32),
            pltpu.VMEM((2, CB, H), jnp.bfloat16),
            pltpu.VMEM((2, RPC + 16, H), jnp.bfloat16),
            pltpu.VMEM((16, 16), jnp.float32),
            pltpu.VMEM((2, RPC + 32), jnp.float32),
            pltpu.SemaphoreType.DMA((2,)),
        ],
        compiler_params=pltpu.CompilerParams(needs_layout_passes=False,
                                             use_tc_tiling_on_sc=False),
    )(center_flat, ctx_flat, u16, v16)


def _loss_body(s_ref, lab_ref, m_ref, out_ref):
    s = s_ref[...]
    lab = lab_ref[...]
    m = m_ref[...]
    per = jnp.maximum(s, 0.0) - s * lab + jnp.log1p(jnp.exp(-jnp.abs(s)))
    num = jnp.sum(per * m)
    den = jnp.maximum(jnp.sum(m), 1.0)
    out_ref[0, 0] = num / den


def _tc_loss(scores2d, label2d, mask2d):
    return pl.pallas_call(
        _loss_body,
        out_shape=jax.ShapeDtypeStruct((1, 1), jnp.float32),
        out_specs=pl.BlockSpec(memory_space=pltpu.SMEM),
    )(scores2d, label2d, mask2d)


def kernel(center, context_neg, label, mask, U, V):
    center_flat = center.reshape(B)
    ctx_flat = context_neg.reshape(B * L)
    scores = _sc_scores(center_flat, ctx_flat,
                        U.astype(jnp.bfloat16), V.astype(jnp.bfloat16))
    scores2d = scores.reshape(B * L // 128, 128)
    label2d = label.reshape(B * L // 128, 128)
    mask2d = mask.reshape(B * L // 128, 128)
    return _tc_loss(scores2d, label2d, mask2d).reshape(())


# speedup vs baseline: 2.7948x; 1.2645x over previous
import functools

import jax
import jax.numpy as jnp
from jax import lax
from jax.experimental import pallas as pl
from jax.experimental.pallas import tpu as pltpu
from jax.experimental.pallas import tpu_sc as plsc

VOCAB = 1_000_000
H = 64
B = 16384
L = 20

NC = 2
NS = 16
NW = NC * NS
BPW = B // NW
CB = 32
NCH = BPW // CB
RPC = CB * L
NG = RPC // 128


def _sc_scores_body(ctx_hbm, u_hbm, v_hbm, out_hbm,
                    vidx, urows, vrows, tbuf, sbuf, gsem, usem):
    wid = lax.axis_index("s") * NC + lax.axis_index("c")
    iota16 = lax.iota(jnp.int32, 16)

    def fire(g, slot):
        base = pl.multiple_of(wid * BPW + g * CB, CB)
        off = pl.multiple_of((wid * BPW + g * CB) * L, RPC)
        pltpu.sync_copy(ctx_hbm.at[pl.ds(off, RPC)], vidx.at[slot])
        pltpu.async_copy(u_hbm.at[pl.ds(base, CB)], urows.at[slot],
                         usem.at[slot])
        for j in range(NG):
            pltpu.async_copy(v_hbm.at[vidx.at[slot, pl.ds(j * 128, 128)]],
                             vrows.at[slot, pl.ds(j * 128, 128)],
                             gsem.at[slot])

    def wait_gathers(slot):
        base = pl.multiple_of(0, CB)
        pltpu.make_async_copy(u_hbm.at[pl.ds(base, CB)], urows.at[slot],
                              usem.at[slot]).wait()
        for j in range(NG):
            pltpu.make_async_copy(v_hbm.at[vidx.at[slot, pl.ds(j * 128, 128)]],
                                  vrows.at[slot, pl.ds(j * 128, 128)],
                                  gsem.at[slot]).wait()

    def compute(g, slot):
        def hsum16():
            tot = jnp.zeros((16,), jnp.float32)
            for k in range(16):
                col = jnp.full((16,), k, jnp.int32)
                tot = tot + plsc.load_gather(tbuf, [iota16, col])
            return tot

        def bbody(b, carry):
            ue0 = urows[slot, b, pl.ds(0, 16)]
            uo0 = urows[slot, b, pl.ds(16, 16)]
            ue1 = urows[slot, b, pl.ds(32, 16)]
            uo1 = urows[slot, b, pl.ds(48, 16)]

            def part(r):
                v0e, v0o = plsc.unpack(vrows[slot, r, pl.ds(0, 32)],
                                       format=plsc.PackFormat.INTERLEAVED)
                v1e, v1o = plsc.unpack(vrows[slot, r, pl.ds(32, 32)],
                                       format=plsc.PackFormat.INTERLEAVED)
                return v0e * ue0 + v0o * uo0 + v1e * ue1 + v1o * uo1

            for l in range(16):
                tbuf[l] = part(b * L + l)
            sbuf[slot, pl.ds(b * L, 16)] = hsum16()
            for l in range(16, L):
                tbuf[l - 16] = part(b * L + l)
            sbuf[slot, pl.ds(b * L + 16, 16)] = hsum16()
            return carry

        lax.fori_loop(0, CB, bbody, 0)
        base = pl.multiple_of((wid * BPW + g * CB) * L, RPC)
        pltpu.sync_copy(sbuf.at[slot, pl.ds(0, RPC)],
                        out_hbm.at[pl.ds(base, RPC)])

    fire(0, 0)

    def pair(i, carry):
        for s in (0, 1):
            g = i * 2 + s

            @pl.when(g + 1 < NCH)
            def _():
                fire(g + 1, (s + 1) % 2)

            wait_gathers(s)
            compute(g, s)
        return carry

    lax.fori_loop(0, NCH // 2, pair, 0)


def _sc_scores(ctx_flat, u_prep, v16):
    mesh = plsc.VectorSubcoreMesh(core_axis_name="c", subcore_axis_name="s",
                                  num_cores=NC, num_subcores=NS)
    return pl.kernel(
        _sc_scores_body,
        out_type=jax.ShapeDtypeStruct((B * L,), jnp.float32),
        mesh=mesh,
        scratch_types=[
            pltpu.VMEM((2, RPC), jnp.int32),
            pltpu.VMEM((2, CB, H), jnp.float32),
            pltpu.VMEM((2, RPC + 16, H), jnp.bfloat16),
            pltpu.VMEM((16, 16), jnp.float32),
            pltpu.VMEM((2, RPC + 32), jnp.float32),
            pltpu.SemaphoreType.DMA((2,)),
            pltpu.SemaphoreType.DMA((2,)),
        ],
        compiler_params=pltpu.CompilerParams(needs_layout_passes=False,
                                             use_tc_tiling_on_sc=False),
    )(ctx_flat, u_prep, v16)


def _loss_body(s_ref, lab_ref, m_ref, out_ref):
    s = s_ref[...]
    lab = lab_ref[...]
    m = m_ref[...]
    per = jnp.maximum(s, 0.0) - s * lab + jnp.log1p(jnp.exp(-jnp.abs(s)))
    num = jnp.sum(per * m)
    den = jnp.maximum(jnp.sum(m), 1.0)
    out_ref[0, 0] = num / den


def _tc_loss(scores2d, label2d, mask2d):
    return pl.pallas_call(
        _loss_body,
        out_shape=jax.ShapeDtypeStruct((1, 1), jnp.float32),
        out_specs=pl.BlockSpec(memory_space=pltpu.SMEM),
    )(scores2d, label2d, mask2d)


def kernel(center, context_neg, label, mask, U, V):
    ctx_flat = context_neg.reshape(B * L)
    u_pre = jnp.take(U, center[:, 0], axis=0)
    u_prep = jnp.concatenate(
        [u_pre[:, 0:32:2], u_pre[:, 1:32:2],
         u_pre[:, 32:64:2], u_pre[:, 33:64:2]], axis=1)
    scores = _sc_scores(ctx_flat, u_prep, V.astype(jnp.bfloat16))
    scores2d = scores.reshape(B * L // 128, 128)
    label2d = label.reshape(B * L // 128, 128)
    mask2d = mask.reshape(B * L // 128, 128)
    return _tc_loss(scores2d, label2d, mask2d).reshape(())


# speedup vs baseline: 3.1006x; 1.1094x over previous
import functools

import jax
import jax.numpy as jnp
from jax import lax
from jax.experimental import pallas as pl
from jax.experimental.pallas import tpu as pltpu
from jax.experimental.pallas import tpu_sc as plsc

VOCAB = 1_000_000
H = 64
B = 16384
L = 20

NC = 2
NS = 16
NW = NC * NS
BPW = B // NW
CB = 32
NCH = BPW // CB
RPC = CB * L
NG = RPC // 128


def _sc_scores_body(ctx_hbm, u_hbm, v_hbm, out_hbm,
                    vidx, urows, vrows, tbuf, sbuf, gsem, usem):
    wid = lax.axis_index("s") * NC + lax.axis_index("c")
    iota16 = lax.iota(jnp.int32, 16)

    def fire(g, slot):
        base = pl.multiple_of(wid * BPW + g * CB, CB)
        off = pl.multiple_of((wid * BPW + g * CB) * L, RPC)
        pltpu.sync_copy(ctx_hbm.at[pl.ds(off, RPC)], vidx.at[slot])
        pltpu.async_copy(u_hbm.at[pl.ds(base, CB)], urows.at[slot],
                         usem.at[slot])
        for j in range(NG):
            pltpu.async_copy(v_hbm.at[vidx.at[slot, pl.ds(j * 128, 128)]],
                             vrows.at[slot, pl.ds(j * 128, 128)],
                             gsem.at[slot])

    def wait_gathers(slot):
        base = pl.multiple_of(0, CB)
        pltpu.make_async_copy(u_hbm.at[pl.ds(base, CB)], urows.at[slot],
                              usem.at[slot]).wait()
        for j in range(NG):
            pltpu.make_async_copy(v_hbm.at[vidx.at[slot, pl.ds(j * 128, 128)]],
                                  vrows.at[slot, pl.ds(j * 128, 128)],
                                  gsem.at[slot]).wait()

    def compute(g, slot):
        def hsum16():
            tot = jnp.zeros((16,), jnp.float32)
            for k in range(16):
                col = jnp.full((16,), k, jnp.int32)
                tot = tot + plsc.load_gather(tbuf, [iota16, col])
            return tot

        def bbody(b, carry):
            u0 = urows[slot, b, pl.ds(0, 32)]
            u1 = urows[slot, b, pl.ds(32, 32)]

            def part(r):
                p = (vrows[slot, r, pl.ds(0, 32)] * u0
                     + vrows[slot, r, pl.ds(32, 32)] * u1)
                pe, po = plsc.unpack(p, format=plsc.PackFormat.INTERLEAVED)
                return pe + po

            for l in range(16):
                tbuf[l] = part(b * L + l)
            sbuf[slot, pl.ds(b * L, 16)] = hsum16()
            for l in range(16, L):
                tbuf[l - 16] = part(b * L + l)
            sbuf[slot, pl.ds(b * L + 16, 16)] = hsum16()
            return carry

        lax.fori_loop(0, CB, bbody, 0)
        base = pl.multiple_of((wid * BPW + g * CB) * L, RPC)
        pltpu.sync_copy(sbuf.at[slot, pl.ds(0, RPC)],
                        out_hbm.at[pl.ds(base, RPC)])

    fire(0, 0)

    def pair(i, carry):
        for s in (0, 1):
            g = i * 2 + s

            @pl.when(g + 1 < NCH)
            def _():
                fire(g + 1, (s + 1) % 2)

            wait_gathers(s)
            compute(g, s)
        return carry

    lax.fori_loop(0, NCH // 2, pair, 0)


def _sc_scores(ctx_flat, u_prep, v16):
    mesh = plsc.VectorSubcoreMesh(core_axis_name="c", subcore_axis_name="s",
                                  num_cores=NC, num_subcores=NS)
    return pl.kernel(
        _sc_scores_body,
        out_type=jax.ShapeDtypeStruct((B * L,), jnp.float32),
        mesh=mesh,
        scratch_types=[
            pltpu.VMEM((2, RPC), jnp.int32),
            pltpu.VMEM((2, CB, H), jnp.bfloat16),
            pltpu.VMEM((2, RPC + 16, H), jnp.bfloat16),
            pltpu.VMEM((16, 16), jnp.float32),
            pltpu.VMEM((2, RPC + 32), jnp.float32),
            pltpu.SemaphoreType.DMA((2,)),
            pltpu.SemaphoreType.DMA((2,)),
        ],
        compiler_params=pltpu.CompilerParams(needs_layout_passes=False,
                                             use_tc_tiling_on_sc=False),
    )(ctx_flat, u_prep, v16)


def _loss_body(s_ref, lab_ref, m_ref, out_ref):
    s = s_ref[...]
    lab = lab_ref[...]
    m = m_ref[...]
    per = jnp.maximum(s, 0.0) - s * lab + jnp.log1p(jnp.exp(-jnp.abs(s)))
    num = jnp.sum(per * m)
    den = jnp.maximum(jnp.sum(m), 1.0)
    out_ref[0, 0] = num / den


def _tc_loss(scores2d, label2d, mask2d):
    return pl.pallas_call(
        _loss_body,
        out_shape=jax.ShapeDtypeStruct((1, 1), jnp.float32),
        out_specs=pl.BlockSpec(memory_space=pltpu.SMEM),
    )(scores2d, label2d, mask2d)


def kernel(center, context_neg, label, mask, U, V):
    ctx_flat = context_neg.reshape(B * L)
    u_pre = jnp.take(U, center[:, 0], axis=0).astype(jnp.bfloat16)
    scores = _sc_scores(ctx_flat, u_pre, V.astype(jnp.bfloat16))
    scores2d = scores.reshape(B * L // 128, 128)
    label2d = label.reshape(B * L // 128, 128)
    mask2d = mask.reshape(B * L // 128, 128)
    return _tc_loss(scores2d, label2d, mask2d).reshape(())


# speedup vs baseline: 3.1267x; 1.0084x over previous
import functools

import jax
import jax.numpy as jnp
from jax import lax
from jax.experimental import pallas as pl
from jax.experimental.pallas import tpu as pltpu
from jax.experimental.pallas import tpu_sc as plsc

VOCAB = 1_000_000
H = 64
B = 16384
L = 20

NC = 2
NS = 16
NW = NC * NS
BPW = B // NW
CB = 32
NCH = BPW // CB
RPC = CB * L
NG = RPC // 128


def _sc_scores_body(ctx_hbm, u_hbm, v_hbm, out_hbm,
                    vidx, urows, vrows, tbufa, tbufb, sbuf, gsem, usem):
    wid = lax.axis_index("s") * NC + lax.axis_index("c")
    iota16 = lax.iota(jnp.int32, 16)

    def fire(g, slot):
        base = pl.multiple_of(wid * BPW + g * CB, CB)
        off = pl.multiple_of((wid * BPW + g * CB) * L, RPC)
        pltpu.sync_copy(ctx_hbm.at[pl.ds(off, RPC)], vidx.at[slot])
        pltpu.async_copy(u_hbm.at[pl.ds(base, CB)], urows.at[slot],
                         usem.at[slot])
        for j in range(NG):
            pltpu.async_copy(v_hbm.at[vidx.at[slot, pl.ds(j * 128, 128)]],
                             vrows.at[slot, pl.ds(j * 128, 128)],
                             gsem.at[slot])

    def wait_gathers(slot):
        base = pl.multiple_of(0, CB)
        pltpu.make_async_copy(u_hbm.at[pl.ds(base, CB)], urows.at[slot],
                              usem.at[slot]).wait()
        for j in range(NG):
            pltpu.make_async_copy(v_hbm.at[vidx.at[slot, pl.ds(j * 128, 128)]],
                                  vrows.at[slot, pl.ds(j * 128, 128)],
                                  gsem.at[slot]).wait()

    def compute(g, slot):
        def hsum16(tb):
            gs = [plsc.load_gather(tb, [iota16 * 16 + k]) for k in range(16)]
            while len(gs) > 1:
                gs = [gs[i] + gs[i + 1] for i in range(0, len(gs), 2)]
            return gs[0]

        def bbody(b, carry):
            u0 = urows[slot, b, pl.ds(0, 32)]
            u1 = urows[slot, b, pl.ds(32, 32)]

            def part(r):
                p = (vrows[slot, r, pl.ds(0, 32)] * u0
                     + vrows[slot, r, pl.ds(32, 32)] * u1)
                pe, po = plsc.unpack(p, format=plsc.PackFormat.INTERLEAVED)
                return pe + po

            for l in range(16):
                tbufa[pl.ds(l * 16, 16)] = part(b * L + l)
            for l in range(16, L):
                tbufb[pl.ds((l - 16) * 16, 16)] = part(b * L + l)
            sbuf[slot, pl.ds(b * L, 16)] = hsum16(tbufa)
            sbuf[slot, pl.ds(b * L + 16, 16)] = hsum16(tbufb)
            return carry

        lax.fori_loop(0, CB, bbody, 0)
        base = pl.multiple_of((wid * BPW + g * CB) * L, RPC)
        pltpu.sync_copy(sbuf.at[slot, pl.ds(0, RPC)],
                        out_hbm.at[pl.ds(base, RPC)])

    fire(0, 0)

    def pair(i, carry):
        for s in (0, 1):
            g = i * 2 + s

            @pl.when(g + 1 < NCH)
            def _():
                fire(g + 1, (s + 1) % 2)

            wait_gathers(s)
            compute(g, s)
        return carry

    lax.fori_loop(0, NCH // 2, pair, 0)


def _sc_scores(ctx_flat, u_prep, v16):
    mesh = plsc.VectorSubcoreMesh(core_axis_name="c", subcore_axis_name="s",
                                  num_cores=NC, num_subcores=NS)
    return pl.kernel(
        _sc_scores_body,
        out_type=jax.ShapeDtypeStruct((B * L,), jnp.float32),
        mesh=mesh,
        scratch_types=[
            pltpu.VMEM((2, RPC), jnp.int32),
            pltpu.VMEM((2, CB, H), jnp.bfloat16),
            pltpu.VMEM((2, RPC + 16, H), jnp.bfloat16),
            pltpu.VMEM((256,), jnp.float32),
            pltpu.VMEM((256,), jnp.float32),
            pltpu.VMEM((2, RPC + 32), jnp.float32),
            pltpu.SemaphoreType.DMA((2,)),
            pltpu.SemaphoreType.DMA((2,)),
        ],
        compiler_params=pltpu.CompilerParams(needs_layout_passes=False,
                                             use_tc_tiling_on_sc=False),
    )(ctx_flat, u_prep, v16)


def _loss_body(s_ref, lab_ref, m_ref, out_ref):
    s = s_ref[...]
    lab = lab_ref[...]
    m = m_ref[...]
    per = jnp.maximum(s, 0.0) - s * lab + jnp.log1p(jnp.exp(-jnp.abs(s)))
    num = jnp.sum(per * m)
    den = jnp.maximum(jnp.sum(m), 1.0)
    out_ref[0, 0] = num / den


def _tc_loss(scores2d, label2d, mask2d):
    return pl.pallas_call(
        _loss_body,
        out_shape=jax.ShapeDtypeStruct((1, 1), jnp.float32),
        out_specs=pl.BlockSpec(memory_space=pltpu.SMEM),
    )(scores2d, label2d, mask2d)


def kernel(center, context_neg, label, mask, U, V):
    ctx_flat = context_neg.reshape(B * L)
    u_pre = jnp.take(U, center[:, 0], axis=0).astype(jnp.bfloat16)
    scores = _sc_scores(ctx_flat, u_pre, V.astype(jnp.bfloat16))
    scores2d = scores.reshape(B * L // 128, 128)
    label2d = label.reshape(B * L // 128, 128)
    mask2d = mask.reshape(B * L // 128, 128)
    return _tc_loss(scores2d, label2d, mask2d).reshape(())


# speedup vs baseline: 3.1303x; 1.0011x over previous
import functools

import jax
import jax.numpy as jnp
from jax import lax
from jax.experimental import pallas as pl
from jax.experimental.pallas import tpu as pltpu
from jax.experimental.pallas import tpu_sc as plsc

VOCAB = 1_000_000
H = 64
B = 16384
L = 20

NC = 2
NS = 16
NW = NC * NS
BPW = B // NW
CB = 32
NCH = BPW // CB
RPC = CB * L
NG = RPC // 128


def _sc_scores_body(ctx_hbm, u_hbm, v_hbm, out_hbm,
                    vidx, urows, vrows, tbufa, tbufb, sbuf, gsem, usem):
    wid = lax.axis_index("s") * NC + lax.axis_index("c")
    iota16 = lax.iota(jnp.int32, 16)

    def fire(g, slot):
        base = pl.multiple_of(wid * BPW + g * CB, CB)
        off = pl.multiple_of((wid * BPW + g * CB) * L, RPC)
        pltpu.sync_copy(ctx_hbm.at[pl.ds(off, RPC)], vidx.at[slot])
        pltpu.async_copy(u_hbm.at[pl.ds(base, CB)], urows.at[slot],
                         usem.at[slot])
        for j in range(NG):
            pltpu.async_copy(v_hbm.at[vidx.at[slot, pl.ds(j * 128, 128)]],
                             vrows.at[slot, pl.ds(j * 128, 128)],
                             gsem.at[slot])

    def wait_gathers(slot):
        base = pl.multiple_of(0, CB)
        pltpu.make_async_copy(u_hbm.at[pl.ds(base, CB)], urows.at[slot],
                              usem.at[slot]).wait()
        for j in range(NG):
            pltpu.make_async_copy(v_hbm.at[vidx.at[slot, pl.ds(j * 128, 128)]],
                                  vrows.at[slot, pl.ds(j * 128, 128)],
                                  gsem.at[slot]).wait()

    def compute(g, slot):
        def hsum16(tb):
            gs = [plsc.load_gather(tb, [iota16 * 16 + k]) for k in range(16)]
            while len(gs) > 1:
                gs = [gs[i] + gs[i + 1] for i in range(0, len(gs), 2)]
            return gs[0]

        evens = iota16 * 2

        def bbody(b, carry):
            bsplat = jnp.full((16,), 0, jnp.int32) + b
            ue0 = plsc.load_gather(urows.at[slot], [bsplat, evens])
            uo0 = plsc.load_gather(urows.at[slot], [bsplat, evens + 1])
            ue1 = plsc.load_gather(urows.at[slot], [bsplat, evens + 32])
            uo1 = plsc.load_gather(urows.at[slot], [bsplat, evens + 33])

            def part(r):
                v0e, v0o = plsc.unpack(vrows[slot, r, pl.ds(0, 32)],
                                       format=plsc.PackFormat.INTERLEAVED)
                v1e, v1o = plsc.unpack(vrows[slot, r, pl.ds(32, 32)],
                                       format=plsc.PackFormat.INTERLEAVED)
                return v0e * ue0 + v0o * uo0 + v1e * ue1 + v1o * uo1

            for l in range(16):
                tbufa[pl.ds(l * 16, 16)] = part(b * L + l)
            for l in range(16, L):
                tbufb[pl.ds((l - 16) * 16, 16)] = part(b * L + l)
            sbuf[slot, pl.ds(b * L, 16)] = hsum16(tbufa)
            sbuf[slot, pl.ds(b * L + 16, 16)] = hsum16(tbufb)
            return carry

        lax.fori_loop(0, CB, bbody, 0)
        base = pl.multiple_of((wid * BPW + g * CB) * L, RPC)
        pltpu.sync_copy(sbuf.at[slot, pl.ds(0, RPC)],
                        out_hbm.at[pl.ds(base, RPC)])

    fire(0, 0)

    def pair(i, carry):
        for s in (0, 1):
            g = i * 2 + s

            @pl.when(g + 1 < NCH)
            def _():
                fire(g + 1, (s + 1) % 2)

            wait_gathers(s)
            compute(g, s)
        return carry

    lax.fori_loop(0, NCH // 2, pair, 0)


def _sc_scores(ctx_flat, u_prep, v16):
    mesh = plsc.VectorSubcoreMesh(core_axis_name="c", subcore_axis_name="s",
                                  num_cores=NC, num_subcores=NS)
    return pl.kernel(
        _sc_scores_body,
        out_type=jax.ShapeDtypeStruct((B * L,), jnp.float32),
        mesh=mesh,
        scratch_types=[
            pltpu.VMEM((2, RPC), jnp.int32),
            pltpu.VMEM((2, CB, H), jnp.float32),
            pltpu.VMEM((2, RPC + 16, H), jnp.bfloat16),
            pltpu.VMEM((256,), jnp.float32),
            pltpu.VMEM((256,), jnp.float32),
            pltpu.VMEM((2, RPC + 32), jnp.float32),
            pltpu.SemaphoreType.DMA((2,)),
            pltpu.SemaphoreType.DMA((2,)),
        ],
        compiler_params=pltpu.CompilerParams(needs_layout_passes=False,
                                             use_tc_tiling_on_sc=False),
    )(ctx_flat, u_prep, v16)


def _loss_body(s_ref, lab_ref, m_ref, out_ref):
    s = s_ref[...]
    lab = lab_ref[...]
    m = m_ref[...]
    per = jnp.maximum(s, 0.0) - s * lab + jnp.log1p(jnp.exp(-jnp.abs(s)))
    num = jnp.sum(per * m)
    den = jnp.maximum(jnp.sum(m), 1.0)
    out_ref[0, 0] = num / den


def _tc_loss(scores2d, label2d, mask2d):
    return pl.pallas_call(
        _loss_body,
        out_shape=jax.ShapeDtypeStruct((1, 1), jnp.float32),
        out_specs=pl.BlockSpec(memory_space=pltpu.SMEM),
    )(scores2d, label2d, mask2d)


def kernel(center, context_neg, label, mask, U, V):
    ctx_flat = context_neg.reshape(B * L)
    u_pre = jnp.take(U, center[:, 0], axis=0)
    scores = _sc_scores(ctx_flat, u_pre, V.astype(jnp.bfloat16))
    scores2d = scores.reshape(B * L // 128, 128)
    label2d = label.reshape(B * L // 128, 128)
    mask2d = mask.reshape(B * L // 128, 128)
    return _tc_loss(scores2d, label2d, mask2d).reshape(())
